# baseline (speedup 1.0000x reference)
import functools

import jax
import jax.numpy as jnp
from jax import lax
from jax.experimental import pallas as pl
from jax.experimental.pallas import tpu as pltpu
from jax.experimental.pallas import tpu_sc as plsc

N = 10000
D = 256
H = 128
E = 160000
NPAD = 10112
ZR = NPAD // 16
CH = 128
EPT = 10112
EPAD = EPT * 16
NCH = EPT // CH

_PREC = lax.Precision.HIGHEST
_DN = (((1,), (0,)), ((), ()))

RB = 2000


def _dot(a, b):
    return lax.dot_general(a, b, _DN, precision=_PREC,
                           preferred_element_type=jnp.float32)


def _mm1_body(x_ref, w_ref, o_ref):
    res = _dot(x_ref[...], w_ref[...])
    o_ref[0] = res[:, :H]
    o_ref[1] = res[:, H:]


def _mid_body(agg_ref, b_ref, w_ref, o_ref):
    h = jnp.concatenate([agg_ref[0], agg_ref[1]], axis=1)
    h = jnp.maximum(h + b_ref[...], 0.0)
    res = _dot(h, w_ref[...])
    o_ref[0] = res[:, :H]
    o_ref[1] = res[:, H:]


def _ff_body(agg_ref, bg_ref, w1_ref, b1_ref, w2_ref, b2_ref,
             w3_ref, b3_ref, ws_ref, bs_ref, o_ref):
    h = jnp.concatenate([agg_ref[0], agg_ref[1]], axis=1) + bg_ref[...]
    z = jnp.maximum(_dot(h, w1_ref[...]) + b1_ref[...], 0.0)
    z = jnp.maximum(_dot(z, w2_ref[...]) + b2_ref[...], 0.0)
    z = jnp.maximum(_dot(z, w3_ref[...]) + b3_ref[...], 0.0)
    o_ref[...] = z + _dot(h, ws_ref[...]) + bs_ref[...]


def _mm1(x, w):
    return pl.pallas_call(
        _mm1_body,
        grid=(N // RB,),
        in_specs=[
            pl.BlockSpec((RB, D), lambda i: (i, 0)),
            pl.BlockSpec((D, D), lambda i: (0, 0)),
        ],
        out_specs=pl.BlockSpec((2, RB, H), lambda i: (0, i, 0)),
        out_shape=jax.ShapeDtypeStruct((2, N, H), jnp.float32),
    )(x, w)


def _mid(agg, b, w):
    return pl.pallas_call(
        _mid_body,
        grid=(N // RB,),
        in_specs=[
            pl.BlockSpec((2, RB, H), lambda i: (0, i, 0)),
            pl.BlockSpec((1, D), lambda i: (0, 0)),
            pl.BlockSpec((D, D), lambda i: (0, 0)),
        ],
        out_specs=pl.BlockSpec((2, RB, H), lambda i: (0, i, 0)),
        out_shape=jax.ShapeDtypeStruct((2, N, H), jnp.float32),
    )(agg, b, w)


def _ff(agg, bg, w1, b1, w2, b2, w3, b3, ws, bs):
    wspec = pl.BlockSpec((D, D), lambda i: (0, 0))
    bspec = pl.BlockSpec((1, D), lambda i: (0, 0))
    return pl.pallas_call(
        _ff_body,
        grid=(N // RB,),
        in_specs=[
            pl.BlockSpec((2, RB, H), lambda i: (0, i, 0)),
            bspec, wspec, bspec, wspec, bspec, wspec, bspec, wspec, bspec,
        ],
        out_specs=pl.BlockSpec((RB, D), lambda i: (i, 0)),
        out_shape=jax.ShapeDtypeStruct((N, D), jnp.float32),
    )(agg, bg, w1, b1, w2, b2, w3, b3, ws, bs)


NB = 3
PD = NB - 1


@functools.lru_cache(maxsize=1)
def _make_seg_sum():
    mesh = plsc.VectorSubcoreMesh(core_axis_name="c", subcore_axis_name="s")

    @functools.partial(
        pl.kernel,
        mesh=mesh,
        out_type=jax.ShapeDtypeStruct((2, NPAD, H), jnp.float32),
        scratch_types=[
            pltpu.VMEM((NB, 2, CH), jnp.int32),
            pltpu.VMEM((NB, CH, H), jnp.float32),
            pltpu.VMEM_SHARED((NPAD, H), jnp.float32),
            pltpu.SemaphoreType.DMA((NB,)),
            pltpu.SemaphoreType.DMA((NB,)),
            pltpu.SemaphoreType.DMA((NB,)),
        ],
    )
    def _seg_sum(table_hbm, idx_hbm, zeros_hbm, out_hbm,
                 idx_v, rows_v, acc, sem_i, sem_g, sem_s):
        c = lax.axis_index("c")
        s = lax.axis_index("s")
        pltpu.sync_copy(zeros_hbm, acc.at[pl.ds(s * ZR, ZR)])
        plsc.subcore_barrier()

        my_idx = idx_hbm.at[c].at[s]

        def idx_load(j, b):
            pltpu.async_copy(my_idx.at[j], idx_v.at[b], sem_i.at[b])

        def wait_i(b):
            pltpu.make_async_copy(my_idx.at[0], idx_v.at[b], sem_i.at[b]).wait()

        def gather(b):
            pltpu.async_copy(table_hbm.at[idx_v.at[b].at[0]], rows_v.at[b],
                             sem_g.at[b])

        def wait_g(b):
            pltpu.make_async_copy(table_hbm.at[pl.ds(0, CH)], rows_v.at[b],
                                  sem_g.at[b]).wait()

        def scatter(b):
            pltpu.async_copy(rows_v.at[b], acc.at[idx_v.at[b].at[1]],
                             sem_s.at[b], add=True)

        def wait_s(b):
            pltpu.make_async_copy(rows_v.at[b], acc.at[pl.ds(0, CH)],
                                  sem_s.at[b]).wait()

        def step(j, *, ws, di, dg, dsc):
            b = j % NB
            if ws:
                wait_s(b)
            if di:
                idx_load(j, b)
            if dg:
                bg = (j - 1) % NB
                wait_i(bg); gather(bg)
            if dsc:
                bs = (j - PD) % NB
                wait_g(bs); scatter(bs)

        for j in range(NB):
            step(j, ws=False, di=True, dg=(j >= 1), dsc=(j >= PD))

        def body(j, carry):
            step(j, ws=True, di=True, dg=True, dsc=True)
            return carry

        lax.fori_loop(NB, NCH, body, 0)

        for j in range(NCH, NCH + PD + 1):
            step(j, ws=True, di=False, dg=(j <= NCH), dsc=(j < NCH + PD))

        plsc.subcore_barrier()
        pltpu.sync_copy(acc.at[pl.ds(s * ZR, ZR)],
                        out_hbm.at[c].at[pl.ds(s * ZR, ZR)])

    return _seg_sum


def _seg_sum_call(table2, idx, zeros):
    return _make_seg_sum()(table2, idx, zeros)


def kernel(x, edge_index, W_g1, b_g1, W_g2, b_g2,
           W_f1, b_f1, W_f2, b_f2, W_f3, b_f3, W_fs, b_fs):
    src = edge_index[0].astype(jnp.int32)
    dst = edge_index[1].astype(jnp.int32)
    src, dst = lax.sort([src, dst], num_keys=1, is_stable=False)
    pad = EPAD - E
    src_p = jnp.concatenate([src, jnp.zeros((pad,), jnp.int32)])
    dst_p = jnp.concatenate([dst, jnp.full((pad,), N, jnp.int32)])
    idx = jnp.stack([
        jnp.stack([src_p, dst_p]),
        jnp.stack([src_p + N, dst_p]),
    ])
    idx = idx.transpose(0, 2, 1).reshape(2, 16, NCH, CH, 2)
    idx = idx.transpose(0, 1, 2, 4, 3)
    zeros = jnp.zeros((ZR, H), jnp.float32)

    b_g1r = b_g1.reshape(1, D)
    b_g2r = b_g2.reshape(1, D)

    support1 = _mm1(x, W_g1).reshape(2 * N, H)
    agg1 = _seg_sum_call(support1, idx, zeros)
    support2 = _mid(agg1, b_g1r, W_g2).reshape(2 * N, H)
    agg2 = _seg_sum_call(support2, idx, zeros)
    out = _ff(agg2, b_g2r,
              W_f1, b_f1.reshape(1, D), W_f2, b_f2.reshape(1, D),
              W_f3, b_f3.reshape(1, D), W_fs, b_fs.reshape(1, D))
    return out


# speedup vs baseline: 1.5656x; 1.5656x over previous
import functools

import jax
import jax.numpy as jnp
from jax import lax
from jax.experimental import pallas as pl
from jax.experimental.pallas import tpu as pltpu
from jax.experimental.pallas import tpu_sc as plsc

N = 10000
D = 256
H = 128
E = 160000
NPAD = 10112
ZR = NPAD // 16
CH = 128
EPT = 10112
EPAD = EPT * 16
NCH = EPT // CH

_PREC = lax.Precision.DEFAULT
_DN = (((1,), (0,)), ((), ()))

RB = 2000


def _dot(a, b):
    return lax.dot_general(a, b, _DN, precision=_PREC,
                           preferred_element_type=jnp.float32)


def _mm1_body(x_ref, w_ref, o_ref):
    res = _dot(x_ref[...], w_ref[...])
    o_ref[0] = res[:, :H]
    o_ref[1] = res[:, H:]


def _mid_body(agg_ref, b_ref, w_ref, o_ref):
    h = jnp.concatenate([agg_ref[0], agg_ref[1]], axis=1)
    h = jnp.maximum(h + b_ref[...], 0.0)
    res = _dot(h, w_ref[...])
    o_ref[0] = res[:, :H]
    o_ref[1] = res[:, H:]


def _ff_body(agg_ref, bg_ref, w1_ref, b1_ref, w2_ref, b2_ref,
             w3_ref, b3_ref, ws_ref, bs_ref, o_ref):
    h = jnp.concatenate([agg_ref[0], agg_ref[1]], axis=1) + bg_ref[...]
    z = jnp.maximum(_dot(h, w1_ref[...]) + b1_ref[...], 0.0)
    z = jnp.maximum(_dot(z, w2_ref[...]) + b2_ref[...], 0.0)
    z = jnp.maximum(_dot(z, w3_ref[...]) + b3_ref[...], 0.0)
    o_ref[...] = z + _dot(h, ws_ref[...]) + bs_ref[...]


def _mm1(x, w):
    return pl.pallas_call(
        _mm1_body,
        grid=(N // RB,),
        in_specs=[
            pl.BlockSpec((RB, D), lambda i: (i, 0)),
            pl.BlockSpec((D, D), lambda i: (0, 0)),
        ],
        out_specs=pl.BlockSpec((2, RB, H), lambda i: (0, i, 0)),
        out_shape=jax.ShapeDtypeStruct((2, N, H), jnp.float32),
    )(x, w)


def _mid(agg, b, w):
    return pl.pallas_call(
        _mid_body,
        grid=(N // RB,),
        in_specs=[
            pl.BlockSpec((2, RB, H), lambda i: (0, i, 0)),
            pl.BlockSpec((1, D), lambda i: (0, 0)),
            pl.BlockSpec((D, D), lambda i: (0, 0)),
        ],
        out_specs=pl.BlockSpec((2, RB, H), lambda i: (0, i, 0)),
        out_shape=jax.ShapeDtypeStruct((2, N, H), jnp.float32),
    )(agg, b, w)


def _ff(agg, bg, w1, b1, w2, b2, w3, b3, ws, bs):
    wspec = pl.BlockSpec((D, D), lambda i: (0, 0))
    bspec = pl.BlockSpec((1, D), lambda i: (0, 0))
    return pl.pallas_call(
        _ff_body,
        grid=(N // RB,),
        in_specs=[
            pl.BlockSpec((2, RB, H), lambda i: (0, i, 0)),
            bspec, wspec, bspec, wspec, bspec, wspec, bspec, wspec, bspec,
        ],
        out_specs=pl.BlockSpec((RB, D), lambda i: (i, 0)),
        out_shape=jax.ShapeDtypeStruct((N, D), jnp.float32),
    )(agg, bg, w1, b1, w2, b2, w3, b3, ws, bs)


NB = 3
PD = NB - 1


@functools.lru_cache(maxsize=1)
def _make_seg_sum():
    mesh = plsc.VectorSubcoreMesh(core_axis_name="c", subcore_axis_name="s")

    @functools.partial(
        pl.kernel,
        mesh=mesh,
        out_type=jax.ShapeDtypeStruct((2, NPAD, H), jnp.float32),
        scratch_types=[
            pltpu.VMEM((NB, 2, CH), jnp.int32),
            pltpu.VMEM((NB, CH, H), jnp.float32),
            pltpu.VMEM_SHARED((NPAD, H), jnp.float32),
            pltpu.SemaphoreType.DMA((NB,)),
            pltpu.SemaphoreType.DMA((NB,)),
            pltpu.SemaphoreType.DMA((NB,)),
            pltpu.SemaphoreType.DMA,
        ],
    )
    def _seg_sum(table_hbm, idx_hbm, zeros_hbm, out_hbm,
                 idx_v, rows_v, acc, sem_i, sem_g, sem_s, sem_z):
        c = lax.axis_index("c")
        s = lax.axis_index("s")
        pltpu.async_copy(zeros_hbm, acc.at[pl.ds(s * ZR, ZR)], sem_z)

        my_idx = idx_hbm.at[c].at[s]

        def idx_load(j, b):
            pltpu.async_copy(my_idx.at[j], idx_v.at[b], sem_i.at[b])

        def wait_i(b):
            pltpu.make_async_copy(my_idx.at[0], idx_v.at[b], sem_i.at[b]).wait()

        def gather(b):
            pltpu.async_copy(table_hbm.at[idx_v.at[b].at[0]], rows_v.at[b],
                             sem_g.at[b])

        def wait_g(b):
            pltpu.make_async_copy(table_hbm.at[pl.ds(0, CH)], rows_v.at[b],
                                  sem_g.at[b]).wait()

        def scatter(b):
            pltpu.async_copy(rows_v.at[b], acc.at[idx_v.at[b].at[1]],
                             sem_s.at[b], add=True)

        def wait_s(b):
            pltpu.make_async_copy(rows_v.at[b], acc.at[pl.ds(0, CH)],
                                  sem_s.at[b]).wait()

        def step(j, *, ws, di, dg, dsc):
            b = j % NB
            if ws:
                wait_s(b)
            if di:
                idx_load(j, b)
            if dg:
                bg = (j - 1) % NB
                wait_i(bg); gather(bg)
            if dsc:
                bs = (j - PD) % NB
                wait_g(bs); scatter(bs)

        for j in range(NB):
            if j == PD:
                pltpu.make_async_copy(zeros_hbm, acc.at[pl.ds(0, ZR)],
                                      sem_z).wait()
                plsc.subcore_barrier()
            step(j, ws=False, di=True, dg=(j >= 1), dsc=(j >= PD))

        def body(j, carry):
            step(j, ws=True, di=True, dg=True, dsc=True)
            return carry

        lax.fori_loop(NB, NCH, body, 0)

        for j in range(NCH, NCH + PD + 1):
            step(j, ws=True, di=False, dg=(j <= NCH), dsc=(j < NCH + PD))

        plsc.subcore_barrier()
        pltpu.sync_copy(acc.at[pl.ds(s * ZR, ZR)],
                        out_hbm.at[c].at[pl.ds(s * ZR, ZR)])

    return _seg_sum


def _seg_sum_call(table2, idx, zeros):
    return _make_seg_sum()(table2, idx, zeros)


def kernel(x, edge_index, W_g1, b_g1, W_g2, b_g2,
           W_f1, b_f1, W_f2, b_f2, W_f3, b_f3, W_fs, b_fs):
    src = edge_index[0].astype(jnp.int32)
    dst = edge_index[1].astype(jnp.int32)
    pad = EPAD - E
    src_p = jnp.concatenate([src, jnp.zeros((pad,), jnp.int32)])
    dst_p = jnp.concatenate([dst, jnp.full((pad,), N, jnp.int32)])
    idx = jnp.stack([
        jnp.stack([src_p, dst_p]),
        jnp.stack([src_p + N, dst_p]),
    ])
    idx = idx.transpose(0, 2, 1).reshape(2, 16, NCH, CH, 2)
    idx = idx.transpose(0, 1, 2, 4, 3)
    zeros = jnp.zeros((ZR, H), jnp.float32)

    b_g1r = b_g1.reshape(1, D)
    b_g2r = b_g2.reshape(1, D)

    support1 = _mm1(x, W_g1).reshape(2 * N, H)
    agg1 = _seg_sum_call(support1, idx, zeros)
    support2 = _mid(agg1, b_g1r, W_g2).reshape(2 * N, H)
    agg2 = _seg_sum_call(support2, idx, zeros)
    out = _ff(agg2, b_g2r,
              W_f1, b_f1.reshape(1, D), W_f2, b_f2.reshape(1, D),
              W_f3, b_f3.reshape(1, D), W_fs, b_fs.reshape(1, D))
    return out
